# baseline (speedup 1.0000x reference)
import functools

import jax
import jax.numpy as jnp
from jax import lax
from jax.experimental import pallas as pl
from jax.experimental.pallas import tpu as pltpu
from jax.experimental.pallas import tpu_sc as plsc

N = 10000
E = 160000
D = 256
HF = 128
NC = 2
NS = 16
EP = 163840
EPT = EP // NS
IRT = EPT // 128
ERW = EP // 128
NCHUNK = EPT // 256
RPT = 624
TAIL = N - NS * RPT
DUM = 128
RZD = 632
ZTAIL = N + DUM - NS * RZD

_mesh = plsc.VectorSubcoreMesh(core_axis_name="c", subcore_axis_name="s")


@functools.partial(
    pl.kernel,
    out_type=jax.ShapeDtypeStruct((2 * N, 16), jnp.float32),
    mesh=_mesh,
    scratch_types=[
        pltpu.VMEM((40, 128), jnp.int32),
        pltpu.VMEM((128, 16), jnp.float32),
        pltpu.VMEM_SHARED((N + DUM, 16), jnp.float32),
        pltpu.SemaphoreType.DMA,
    ],
    compiler_params=pltpu.CompilerParams(use_tc_tiling_on_sc=False),
)
def _deg_kernel(dst2, ones_hbm, zeros_hbm, out, di, ones_v, accum, ssem):
    c = lax.axis_index("c")
    s = lax.axis_index("s")
    pltpu.sync_copy(zeros_hbm.at[pl.ds(0, RZD)], accum.at[pl.ds(s * RZD, RZD)])

    @pl.when(s == NS - 1)
    def _():
        pltpu.sync_copy(zeros_hbm.at[pl.ds(0, ZTAIL)],
                        accum.at[pl.ds(NS * RZD, ZTAIL)])

    pltpu.sync_copy(ones_hbm, ones_v)
    base_row = c * (ERW // 2) + s * 40
    pltpu.sync_copy(dst2.at[pl.ds(base_row, 40)], di)
    plsc.subcore_barrier()

    def body(i, carry):
        pltpu.sync_copy(ones_v, accum.at[di.at[i]], add=True)
        return carry

    lax.fori_loop(0, 40, body, 0)
    plsc.subcore_barrier()
    pltpu.sync_copy(accum.at[pl.ds(s * RPT, RPT)],
                    out.at[pl.ds(c * N + s * RPT, RPT)])

    @pl.when(s == NS - 1)
    def _():
        pltpu.sync_copy(accum.at[pl.ds(NS * RPT, TAIL)],
                        out.at[pl.ds(c * N + NS * RPT, TAIL)])


@functools.partial(
    pl.kernel,
    out_type=jax.ShapeDtypeStruct((NC * N, HF), jnp.bfloat16),
    mesh=_mesh,
    scratch_types=[
        pltpu.VMEM((IRT, 128), jnp.int32),
        pltpu.VMEM((IRT, 128), jnp.int32),
        pltpu.VMEM((256, HF), jnp.bfloat16),
        pltpu.VMEM((256, HF), jnp.bfloat16),
        pltpu.VMEM_SHARED((N + DUM, HF), jnp.bfloat16),
        pltpu.SemaphoreType.DMA,
        pltpu.SemaphoreType.DMA,
    ],
    compiler_params=pltpu.CompilerParams(use_tc_tiling_on_sc=False),
)
def _spmm_kernel(u, srcp2, dst2, zer, out, si, di, rows0, rows1, accum,
                 gs0, gs1):
    c = lax.axis_index("c")
    s = lax.axis_index("s")
    irow = s * IRT
    pltpu.sync_copy(dst2.at[pl.ds(irow, IRT)], di)
    pltpu.sync_copy(srcp2.at[pl.ds(c * ERW + irow, IRT)], si)

    pltpu.sync_copy(u.at[pl.ds(c * N + s * RPT, RPT)],
                    accum.at[pl.ds(s * RPT, RPT)])

    @pl.when(s == NS - 1)
    def _():
        pltpu.sync_copy(u.at[pl.ds(c * N + NS * RPT, TAIL)],
                        accum.at[pl.ds(NS * RPT, TAIL)])

    @pl.when(s == 0)
    def _():
        pltpu.sync_copy(zer, accum.at[pl.ds(N, DUM)])

    plsc.subcore_barrier()

    def gather(k, buf, sem):
        r = 2 * k
        pltpu.async_copy(u.at[si.at[r]], buf.at[pl.ds(0, 128)], sem)
        pltpu.async_copy(u.at[si.at[r + 1]], buf.at[pl.ds(128, 128)], sem)

    def gwait(buf, sem):
        pltpu.make_async_copy(u.at[si.at[0]], buf.at[pl.ds(0, 128)], sem).wait()
        pltpu.make_async_copy(u.at[si.at[0]], buf.at[pl.ds(128, 128)], sem).wait()

    def scatter(k, buf):
        r = 2 * k
        pltpu.sync_copy(buf.at[pl.ds(0, 128)], accum.at[di.at[r]], add=True)
        pltpu.sync_copy(buf.at[pl.ds(128, 128)], accum.at[di.at[r + 1]],
                        add=True)

    gather(0, rows0, gs0)

    def body(t, carry):
        gather(2 * t + 1, rows1, gs1)
        gwait(rows0, gs0)
        scatter(2 * t, rows0)

        @pl.when(t < NCHUNK // 2 - 1)
        def _():
            gather(2 * t + 2, rows0, gs0)

        gwait(rows1, gs1)
        scatter(2 * t + 1, rows1)
        return carry

    lax.fori_loop(0, NCHUNK // 2, body, 0)
    plsc.subcore_barrier()
    pltpu.sync_copy(accum.at[pl.ds(s * RPT, RPT)],
                    out.at[pl.ds(c * N + s * RPT, RPT)])

    @pl.when(s == NS - 1)
    def _():
        pltpu.sync_copy(accum.at[pl.ds(NS * RPT, TAIL)],
                        out.at[pl.ds(c * N + NS * RPT, TAIL)])


BN = 1000


def _dis_from(dp_ref):
    deg = dp_ref[0, :, 0:1] + dp_ref[1, :, 0:1] + 1.0
    return lax.rsqrt(deg)


def _store_halves(u_ref, u):
    ub = u.astype(jnp.bfloat16)
    u_ref[0] = ub[:, 0:HF]
    u_ref[1] = ub[:, HF:D]


def _tc_pre_body(x_ref, q_ref, dp_ref, w_ref, u_ref, h1_ref):
    h1 = q_ref[...] * x_ref[...]
    dis = _dis_from(dp_ref)
    z = jnp.dot(h1, w_ref[...], preferred_element_type=jnp.float32)
    _store_halves(u_ref, dis * z)
    h1_ref[...] = h1


def _tc_mid_body(g0_ref, g1_ref, dp_ref, h_ref, b_ref, w_ref, u_ref, hn_ref):
    dis = _dis_from(dp_ref)
    g = jnp.concatenate([g0_ref[...], g1_ref[...]], axis=1).astype(jnp.float32)
    conv = dis * g + b_ref[...]
    hn = jnp.maximum(conv + h_ref[...], 0.0)
    z = jnp.dot(hn, w_ref[...], preferred_element_type=jnp.float32)
    _store_halves(u_ref, dis * z)
    hn_ref[...] = hn


def _tc_fin_body(g0_ref, g1_ref, dp_ref, h_ref, b_ref, wc_ref, bc_ref, o_ref):
    dis = _dis_from(dp_ref)
    g = jnp.concatenate([g0_ref[...], g1_ref[...]], axis=1).astype(jnp.float32)
    conv = dis * g + b_ref[...]
    h5 = conv + h_ref[...]
    o_ref[...] = (jnp.dot(h5, wc_ref[...], preferred_element_type=jnp.float32)
                  + bc_ref[...])


def _tc_pre(x, query, dp, W1):
    return pl.pallas_call(
        _tc_pre_body,
        grid=(N // BN,),
        in_specs=[
            pl.BlockSpec((BN, D), lambda i: (i, 0)),
            pl.BlockSpec((BN, D), lambda i: (i, 0)),
            pl.BlockSpec((2, BN, 16), lambda i: (0, i, 0)),
            pl.BlockSpec((D, D), lambda i: (0, 0)),
        ],
        out_specs=[
            pl.BlockSpec((NC, BN, HF), lambda i: (0, i, 0)),
            pl.BlockSpec((BN, D), lambda i: (i, 0)),
        ],
        out_shape=[
            jax.ShapeDtypeStruct((NC, N, HF), jnp.bfloat16),
            jax.ShapeDtypeStruct((N, D), jnp.float32),
        ],
    )(x, query, dp, W1)


def _tc_mid(g, dp, h, b, Wn):
    return pl.pallas_call(
        _tc_mid_body,
        grid=(N // BN,),
        in_specs=[
            pl.BlockSpec((BN, HF), lambda i: (i, 0)),
            pl.BlockSpec((BN, HF), lambda i: (N // BN + i, 0)),
            pl.BlockSpec((2, BN, 16), lambda i: (0, i, 0)),
            pl.BlockSpec((BN, D), lambda i: (i, 0)),
            pl.BlockSpec((1, D), lambda i: (0, 0)),
            pl.BlockSpec((D, D), lambda i: (0, 0)),
        ],
        out_specs=[
            pl.BlockSpec((NC, BN, HF), lambda i: (0, i, 0)),
            pl.BlockSpec((BN, D), lambda i: (i, 0)),
        ],
        out_shape=[
            jax.ShapeDtypeStruct((NC, N, HF), jnp.bfloat16),
            jax.ShapeDtypeStruct((N, D), jnp.float32),
        ],
    )(g, g, dp, h, b, Wn)


def _tc_fin(g, dp, h, b, Wc, bc):
    return pl.pallas_call(
        _tc_fin_body,
        grid=(N // BN,),
        in_specs=[
            pl.BlockSpec((BN, HF), lambda i: (i, 0)),
            pl.BlockSpec((BN, HF), lambda i: (N // BN + i, 0)),
            pl.BlockSpec((2, BN, 16), lambda i: (0, i, 0)),
            pl.BlockSpec((BN, D), lambda i: (i, 0)),
            pl.BlockSpec((1, D), lambda i: (0, 0)),
            pl.BlockSpec((D, 64), lambda i: (0, 0)),
            pl.BlockSpec((1, 64), lambda i: (0, 0)),
        ],
        out_specs=pl.BlockSpec((BN, 64), lambda i: (i, 0)),
        out_shape=jax.ShapeDtypeStruct((N, 64), jnp.float32),
    )(g, g, dp, h, b, Wc, bc)


def kernel(x, edge_index, query, W1, b1, W2, b2, W3, b3, W4, b4, Wc, bc):
    src = edge_index[0]
    dst = edge_index[1]
    pad = EP - E
    ar = jnp.arange(pad, dtype=jnp.int32) % DUM
    srcf = jnp.concatenate([src, ar])
    dstf = jnp.concatenate([dst, N + ar])
    srcp2 = jnp.concatenate([srcf, srcf + N]).reshape(-1, 128)
    dst2 = dstf.reshape(-1, 128)
    ones16 = jnp.ones((128, 16), jnp.float32)
    zer16 = jnp.zeros((RZD, 16), jnp.float32)
    zerh = jnp.zeros((DUM, HF), jnp.bfloat16)

    dp = _deg_kernel(dst2, ones16, zer16).reshape(2, N, 16)

    U, h = _tc_pre(x, query, dp, W1)
    g = _spmm_kernel(U.reshape(NC * N, HF), srcp2, dst2, zerh)
    U, h = _tc_mid(g, dp, h, b1.reshape(1, D), W2)
    g = _spmm_kernel(U.reshape(NC * N, HF), srcp2, dst2, zerh)
    U, h = _tc_mid(g, dp, h, b2.reshape(1, D), W3)
    g = _spmm_kernel(U.reshape(NC * N, HF), srcp2, dst2, zerh)
    U, h = _tc_mid(g, dp, h, b3.reshape(1, D), W4)
    g = _spmm_kernel(U.reshape(NC * N, HF), srcp2, dst2, zerh)
    return _tc_fin(g, dp, h, b4.reshape(1, D), Wc,
                   bc.reshape(1, 64))


# speedup vs baseline: 1.0143x; 1.0143x over previous
import functools

import jax
import jax.numpy as jnp
from jax import lax
from jax.experimental import pallas as pl
from jax.experimental.pallas import tpu as pltpu
from jax.experimental.pallas import tpu_sc as plsc

N = 10000
E = 160000
D = 256
HF = 128
NC = 2
NS = 16
EP = 163840
EPT = EP // NS
IRT = EPT // 128
ERW = EP // 128
NCHUNK = EPT // 256
RPT = 624
TAIL = N - NS * RPT
DUM = 128
RZD = 632
ZTAIL = N + DUM - NS * RZD

_mesh = plsc.VectorSubcoreMesh(core_axis_name="c", subcore_axis_name="s")


@functools.partial(
    pl.kernel,
    out_type=jax.ShapeDtypeStruct((2 * N, 16), jnp.float32),
    mesh=_mesh,
    scratch_types=[
        pltpu.VMEM((40, 128), jnp.int32),
        pltpu.VMEM((128, 16), jnp.float32),
        pltpu.VMEM_SHARED((N + DUM, 16), jnp.float32),
        pltpu.SemaphoreType.DMA,
    ],
    compiler_params=pltpu.CompilerParams(use_tc_tiling_on_sc=False),
)
def _deg_kernel(dst2, ones_hbm, zeros_hbm, out, di, ones_v, accum, ssem):
    c = lax.axis_index("c")
    s = lax.axis_index("s")
    pltpu.sync_copy(zeros_hbm.at[pl.ds(0, RZD)], accum.at[pl.ds(s * RZD, RZD)])

    @pl.when(s == NS - 1)
    def _():
        pltpu.sync_copy(zeros_hbm.at[pl.ds(0, ZTAIL)],
                        accum.at[pl.ds(NS * RZD, ZTAIL)])

    pltpu.sync_copy(ones_hbm, ones_v)
    base_row = c * (ERW // 2) + s * 40
    pltpu.sync_copy(dst2.at[pl.ds(base_row, 40)], di)
    plsc.subcore_barrier()

    def body(i, carry):
        pltpu.sync_copy(ones_v, accum.at[di.at[i]], add=True)
        return carry

    lax.fori_loop(0, 40, body, 0)
    plsc.subcore_barrier()
    pltpu.sync_copy(accum.at[pl.ds(s * RPT, RPT)],
                    out.at[pl.ds(c * N + s * RPT, RPT)])

    @pl.when(s == NS - 1)
    def _():
        pltpu.sync_copy(accum.at[pl.ds(NS * RPT, TAIL)],
                        out.at[pl.ds(c * N + NS * RPT, TAIL)])


@functools.partial(
    pl.kernel,
    out_type=jax.ShapeDtypeStruct((NC * N, HF), jnp.bfloat16),
    mesh=_mesh,
    scratch_types=[
        pltpu.VMEM((IRT, 128), jnp.int32),
        pltpu.VMEM((IRT, 128), jnp.int32),
        pltpu.VMEM((256, HF), jnp.bfloat16),
        pltpu.VMEM((256, HF), jnp.bfloat16),
        pltpu.VMEM_SHARED((N + DUM, HF), jnp.bfloat16),
        pltpu.SemaphoreType.DMA,
        pltpu.SemaphoreType.DMA,
    ],
    compiler_params=pltpu.CompilerParams(use_tc_tiling_on_sc=False),
)
def _spmm_kernel(u, srcp2, dst2, zer, out, si, di, rows0, rows1, accum,
                 gs0, gs1):
    c = lax.axis_index("c")
    s = lax.axis_index("s")
    irow = s * IRT
    pltpu.sync_copy(dst2.at[pl.ds(irow, IRT)], di)
    pltpu.sync_copy(srcp2.at[pl.ds(c * ERW + irow, IRT)], si)

    pltpu.sync_copy(u.at[pl.ds(c * N + s * RPT, RPT)],
                    accum.at[pl.ds(s * RPT, RPT)])

    @pl.when(s == NS - 1)
    def _():
        pltpu.sync_copy(u.at[pl.ds(c * N + NS * RPT, TAIL)],
                        accum.at[pl.ds(NS * RPT, TAIL)])

    @pl.when(s == 0)
    def _():
        pltpu.sync_copy(zer, accum.at[pl.ds(N, DUM)])

    plsc.subcore_barrier()

    def gather(k, buf, sem):
        r = 2 * k
        pltpu.async_copy(u.at[si.at[r]], buf.at[pl.ds(0, 128)], sem)
        pltpu.async_copy(u.at[si.at[r + 1]], buf.at[pl.ds(128, 128)], sem)

    def gwait(buf, sem):
        pltpu.make_async_copy(u.at[si.at[0]], buf.at[pl.ds(0, 128)], sem).wait()
        pltpu.make_async_copy(u.at[si.at[0]], buf.at[pl.ds(128, 128)], sem).wait()

    def scatter(k, buf):
        r = 2 * k
        pltpu.sync_copy(buf.at[pl.ds(0, 128)], accum.at[di.at[r]], add=True)
        pltpu.sync_copy(buf.at[pl.ds(128, 128)], accum.at[di.at[r + 1]],
                        add=True)

    gather(0, rows0, gs0)

    def body(t, carry):
        gather(2 * t + 1, rows1, gs1)
        gwait(rows0, gs0)
        scatter(2 * t, rows0)

        @pl.when(t < NCHUNK // 2 - 1)
        def _():
            gather(2 * t + 2, rows0, gs0)

        gwait(rows1, gs1)
        scatter(2 * t + 1, rows1)
        return carry

    lax.fori_loop(0, NCHUNK // 2, body, 0)
    plsc.subcore_barrier()
    pltpu.sync_copy(accum.at[pl.ds(s * RPT, RPT)],
                    out.at[pl.ds(c * N + s * RPT, RPT)])

    @pl.when(s == NS - 1)
    def _():
        pltpu.sync_copy(accum.at[pl.ds(NS * RPT, TAIL)],
                        out.at[pl.ds(c * N + NS * RPT, TAIL)])


BN = 2000


def _dis_from(dp_ref):
    deg = dp_ref[0, :, 0:1] + dp_ref[1, :, 0:1] + 1.0
    return lax.rsqrt(deg)


def _store_halves(u_ref, u):
    ub = u.astype(jnp.bfloat16)
    u_ref[0] = ub[:, 0:HF]
    u_ref[1] = ub[:, HF:D]


def _tc_pre_body(x_ref, q_ref, dp_ref, w_ref, u_ref, h1_ref):
    h1 = q_ref[...] * x_ref[...]
    dis = _dis_from(dp_ref)
    z = jnp.dot(h1, w_ref[...], preferred_element_type=jnp.float32)
    _store_halves(u_ref, dis * z)
    h1_ref[...] = h1


def _tc_mid_body(g0_ref, g1_ref, dp_ref, h_ref, b_ref, w_ref, u_ref, hn_ref):
    dis = _dis_from(dp_ref)
    g = jnp.concatenate([g0_ref[...], g1_ref[...]], axis=1).astype(jnp.float32)
    conv = dis * g + b_ref[...]
    hn = jnp.maximum(conv + h_ref[...], 0.0)
    z = jnp.dot(hn, w_ref[...], preferred_element_type=jnp.float32)
    _store_halves(u_ref, dis * z)
    hn_ref[...] = hn


def _tc_fin_body(g0_ref, g1_ref, dp_ref, h_ref, b_ref, wc_ref, bc_ref, o_ref):
    dis = _dis_from(dp_ref)
    g = jnp.concatenate([g0_ref[...], g1_ref[...]], axis=1).astype(jnp.float32)
    conv = dis * g + b_ref[...]
    h5 = conv + h_ref[...]
    o_ref[...] = (jnp.dot(h5, wc_ref[...], preferred_element_type=jnp.float32)
                  + bc_ref[...])


def _tc_pre(x, query, dp, W1):
    return pl.pallas_call(
        _tc_pre_body,
        grid=(N // BN,),
        in_specs=[
            pl.BlockSpec((BN, D), lambda i: (i, 0)),
            pl.BlockSpec((BN, D), lambda i: (i, 0)),
            pl.BlockSpec((2, BN, 16), lambda i: (0, i, 0)),
            pl.BlockSpec((D, D), lambda i: (0, 0)),
        ],
        out_specs=[
            pl.BlockSpec((NC, BN, HF), lambda i: (0, i, 0)),
            pl.BlockSpec((BN, D), lambda i: (i, 0)),
        ],
        out_shape=[
            jax.ShapeDtypeStruct((NC, N, HF), jnp.bfloat16),
            jax.ShapeDtypeStruct((N, D), jnp.float32),
        ],
    )(x, query, dp, W1)


def _tc_mid(g, dp, h, b, Wn):
    return pl.pallas_call(
        _tc_mid_body,
        grid=(N // BN,),
        in_specs=[
            pl.BlockSpec((BN, HF), lambda i: (i, 0)),
            pl.BlockSpec((BN, HF), lambda i: (N // BN + i, 0)),
            pl.BlockSpec((2, BN, 16), lambda i: (0, i, 0)),
            pl.BlockSpec((BN, D), lambda i: (i, 0)),
            pl.BlockSpec((1, D), lambda i: (0, 0)),
            pl.BlockSpec((D, D), lambda i: (0, 0)),
        ],
        out_specs=[
            pl.BlockSpec((NC, BN, HF), lambda i: (0, i, 0)),
            pl.BlockSpec((BN, D), lambda i: (i, 0)),
        ],
        out_shape=[
            jax.ShapeDtypeStruct((NC, N, HF), jnp.bfloat16),
            jax.ShapeDtypeStruct((N, D), jnp.float32),
        ],
    )(g, g, dp, h, b, Wn)


def _tc_fin(g, dp, h, b, Wc, bc):
    return pl.pallas_call(
        _tc_fin_body,
        grid=(N // BN,),
        in_specs=[
            pl.BlockSpec((BN, HF), lambda i: (i, 0)),
            pl.BlockSpec((BN, HF), lambda i: (N // BN + i, 0)),
            pl.BlockSpec((2, BN, 16), lambda i: (0, i, 0)),
            pl.BlockSpec((BN, D), lambda i: (i, 0)),
            pl.BlockSpec((1, D), lambda i: (0, 0)),
            pl.BlockSpec((D, 64), lambda i: (0, 0)),
            pl.BlockSpec((1, 64), lambda i: (0, 0)),
        ],
        out_specs=pl.BlockSpec((BN, 64), lambda i: (i, 0)),
        out_shape=jax.ShapeDtypeStruct((N, 64), jnp.float32),
    )(g, g, dp, h, b, Wc, bc)


def kernel(x, edge_index, query, W1, b1, W2, b2, W3, b3, W4, b4, Wc, bc):
    src = edge_index[0]
    dst = edge_index[1]
    pad = EP - E
    ar = jnp.arange(pad, dtype=jnp.int32) % DUM
    srcf = jnp.concatenate([src, ar])
    dstf = jnp.concatenate([dst, N + ar])
    srcp2 = jnp.concatenate([srcf, srcf + N]).reshape(-1, 128)
    dst2 = dstf.reshape(-1, 128)
    ones16 = jnp.ones((128, 16), jnp.float32)
    zer16 = jnp.zeros((RZD, 16), jnp.float32)
    zerh = jnp.zeros((DUM, HF), jnp.bfloat16)

    dp = _deg_kernel(dst2, ones16, zer16).reshape(2, N, 16)

    U, h = _tc_pre(x, query, dp, W1)
    g = _spmm_kernel(U.reshape(NC * N, HF), srcp2, dst2, zerh)
    U, h = _tc_mid(g, dp, h, b1.reshape(1, D), W2)
    g = _spmm_kernel(U.reshape(NC * N, HF), srcp2, dst2, zerh)
    U, h = _tc_mid(g, dp, h, b2.reshape(1, D), W3)
    g = _spmm_kernel(U.reshape(NC * N, HF), srcp2, dst2, zerh)
    U, h = _tc_mid(g, dp, h, b3.reshape(1, D), W4)
    g = _spmm_kernel(U.reshape(NC * N, HF), srcp2, dst2, zerh)
    return _tc_fin(g, dp, h, b4.reshape(1, D), Wc,
                   bc.reshape(1, 64))


# speedup vs baseline: 1.0185x; 1.0041x over previous
import functools

import jax
import jax.numpy as jnp
from jax import lax
from jax.experimental import pallas as pl
from jax.experimental.pallas import tpu as pltpu
from jax.experimental.pallas import tpu_sc as plsc

N = 10000
E = 160000
D = 256
HF = 128
NC = 2
NS = 16
EP = 163840
EPT = EP // NS
IRT = EPT // 128
ERW = EP // 128
NCHUNK = EPT // 512
RPT = 624
TAIL = N - NS * RPT
DUM = 128
RZD = 632
ZTAIL = N + DUM - NS * RZD

_mesh = plsc.VectorSubcoreMesh(core_axis_name="c", subcore_axis_name="s")


@functools.partial(
    pl.kernel,
    out_type=jax.ShapeDtypeStruct((2 * N, 16), jnp.float32),
    mesh=_mesh,
    scratch_types=[
        pltpu.VMEM((40, 128), jnp.int32),
        pltpu.VMEM((128, 16), jnp.float32),
        pltpu.VMEM_SHARED((N + DUM, 16), jnp.float32),
        pltpu.SemaphoreType.DMA,
    ],
    compiler_params=pltpu.CompilerParams(use_tc_tiling_on_sc=False),
)
def _deg_kernel(dst2, ones_hbm, zeros_hbm, out, di, ones_v, accum, ssem):
    c = lax.axis_index("c")
    s = lax.axis_index("s")
    pltpu.sync_copy(zeros_hbm.at[pl.ds(0, RZD)], accum.at[pl.ds(s * RZD, RZD)])

    @pl.when(s == NS - 1)
    def _():
        pltpu.sync_copy(zeros_hbm.at[pl.ds(0, ZTAIL)],
                        accum.at[pl.ds(NS * RZD, ZTAIL)])

    pltpu.sync_copy(ones_hbm, ones_v)
    base_row = c * (ERW // 2) + s * 40
    pltpu.sync_copy(dst2.at[pl.ds(base_row, 40)], di)
    plsc.subcore_barrier()

    def body(i, carry):
        pltpu.sync_copy(ones_v, accum.at[di.at[i]], add=True)
        return carry

    lax.fori_loop(0, 40, body, 0)
    plsc.subcore_barrier()
    pltpu.sync_copy(accum.at[pl.ds(s * RPT, RPT)],
                    out.at[pl.ds(c * N + s * RPT, RPT)])

    @pl.when(s == NS - 1)
    def _():
        pltpu.sync_copy(accum.at[pl.ds(NS * RPT, TAIL)],
                        out.at[pl.ds(c * N + NS * RPT, TAIL)])


@functools.partial(
    pl.kernel,
    out_type=jax.ShapeDtypeStruct((NC * N, HF), jnp.bfloat16),
    mesh=_mesh,
    scratch_types=[
        pltpu.VMEM((IRT, 128), jnp.int32),
        pltpu.VMEM((IRT, 128), jnp.int32),
        pltpu.VMEM((512, HF), jnp.bfloat16),
        pltpu.VMEM((512, HF), jnp.bfloat16),
        pltpu.VMEM_SHARED((N + DUM, HF), jnp.bfloat16),
        pltpu.SemaphoreType.DMA,
        pltpu.SemaphoreType.DMA,
    ],
    compiler_params=pltpu.CompilerParams(use_tc_tiling_on_sc=False),
)
def _spmm_kernel(u, srcp2, dst2, zer, out, si, di, rows0, rows1, accum,
                 gs0, gs1):
    c = lax.axis_index("c")
    s = lax.axis_index("s")
    irow = s * IRT
    pltpu.sync_copy(dst2.at[pl.ds(irow, IRT)], di)
    pltpu.sync_copy(srcp2.at[pl.ds(c * ERW + irow, IRT)], si)

    pltpu.sync_copy(u.at[pl.ds(c * N + s * RPT, RPT)],
                    accum.at[pl.ds(s * RPT, RPT)])

    @pl.when(s == NS - 1)
    def _():
        pltpu.sync_copy(u.at[pl.ds(c * N + NS * RPT, TAIL)],
                        accum.at[pl.ds(NS * RPT, TAIL)])

    @pl.when(s == 0)
    def _():
        pltpu.sync_copy(zer, accum.at[pl.ds(N, DUM)])

    plsc.subcore_barrier()

    def gather(k, buf, sem):
        r = 4 * k
        for j in range(4):
            pltpu.async_copy(u.at[si.at[r + j]],
                             buf.at[pl.ds(j * 128, 128)], sem)

    def gwait(buf, sem):
        for j in range(4):
            pltpu.make_async_copy(u.at[si.at[0]],
                                  buf.at[pl.ds(j * 128, 128)], sem).wait()

    def scatter(k, buf):
        r = 4 * k
        for j in range(4):
            pltpu.sync_copy(buf.at[pl.ds(j * 128, 128)],
                            accum.at[di.at[r + j]], add=True)

    gather(0, rows0, gs0)

    def body(t, carry):
        gather(2 * t + 1, rows1, gs1)
        gwait(rows0, gs0)
        scatter(2 * t, rows0)

        @pl.when(t < NCHUNK // 2 - 1)
        def _():
            gather(2 * t + 2, rows0, gs0)

        gwait(rows1, gs1)
        scatter(2 * t + 1, rows1)
        return carry

    lax.fori_loop(0, NCHUNK // 2, body, 0)
    plsc.subcore_barrier()
    pltpu.sync_copy(accum.at[pl.ds(s * RPT, RPT)],
                    out.at[pl.ds(c * N + s * RPT, RPT)])

    @pl.when(s == NS - 1)
    def _():
        pltpu.sync_copy(accum.at[pl.ds(NS * RPT, TAIL)],
                        out.at[pl.ds(c * N + NS * RPT, TAIL)])


BN = 2000


def _dis_from(dp_ref):
    deg = dp_ref[0, :, 0:1] + dp_ref[1, :, 0:1] + 1.0
    return lax.rsqrt(deg)


def _store_halves(u_ref, u):
    ub = u.astype(jnp.bfloat16)
    u_ref[0] = ub[:, 0:HF]
    u_ref[1] = ub[:, HF:D]


def _tc_pre_body(x_ref, q_ref, dp_ref, w_ref, u_ref, h1_ref):
    h1 = q_ref[...] * x_ref[...]
    dis = _dis_from(dp_ref)
    z = jnp.dot(h1, w_ref[...], preferred_element_type=jnp.float32)
    _store_halves(u_ref, dis * z)
    h1_ref[...] = h1


def _tc_mid_body(g0_ref, g1_ref, dp_ref, h_ref, b_ref, w_ref, u_ref, hn_ref):
    dis = _dis_from(dp_ref)
    g = jnp.concatenate([g0_ref[...], g1_ref[...]], axis=1).astype(jnp.float32)
    conv = dis * g + b_ref[...]
    hn = jnp.maximum(conv + h_ref[...], 0.0)
    z = jnp.dot(hn, w_ref[...], preferred_element_type=jnp.float32)
    _store_halves(u_ref, dis * z)
    hn_ref[...] = hn


def _tc_fin_body(g0_ref, g1_ref, dp_ref, h_ref, b_ref, wc_ref, bc_ref, o_ref):
    dis = _dis_from(dp_ref)
    g = jnp.concatenate([g0_ref[...], g1_ref[...]], axis=1).astype(jnp.float32)
    conv = dis * g + b_ref[...]
    h5 = conv + h_ref[...]
    o_ref[...] = (jnp.dot(h5, wc_ref[...], preferred_element_type=jnp.float32)
                  + bc_ref[...])


def _tc_pre(x, query, dp, W1):
    return pl.pallas_call(
        _tc_pre_body,
        grid=(N // BN,),
        in_specs=[
            pl.BlockSpec((BN, D), lambda i: (i, 0)),
            pl.BlockSpec((BN, D), lambda i: (i, 0)),
            pl.BlockSpec((2, BN, 16), lambda i: (0, i, 0)),
            pl.BlockSpec((D, D), lambda i: (0, 0)),
        ],
        out_specs=[
            pl.BlockSpec((NC, BN, HF), lambda i: (0, i, 0)),
            pl.BlockSpec((BN, D), lambda i: (i, 0)),
        ],
        out_shape=[
            jax.ShapeDtypeStruct((NC, N, HF), jnp.bfloat16),
            jax.ShapeDtypeStruct((N, D), jnp.float32),
        ],
    )(x, query, dp, W1)


def _tc_mid(g, dp, h, b, Wn):
    return pl.pallas_call(
        _tc_mid_body,
        grid=(N // BN,),
        in_specs=[
            pl.BlockSpec((BN, HF), lambda i: (i, 0)),
            pl.BlockSpec((BN, HF), lambda i: (N // BN + i, 0)),
            pl.BlockSpec((2, BN, 16), lambda i: (0, i, 0)),
            pl.BlockSpec((BN, D), lambda i: (i, 0)),
            pl.BlockSpec((1, D), lambda i: (0, 0)),
            pl.BlockSpec((D, D), lambda i: (0, 0)),
        ],
        out_specs=[
            pl.BlockSpec((NC, BN, HF), lambda i: (0, i, 0)),
            pl.BlockSpec((BN, D), lambda i: (i, 0)),
        ],
        out_shape=[
            jax.ShapeDtypeStruct((NC, N, HF), jnp.bfloat16),
            jax.ShapeDtypeStruct((N, D), jnp.float32),
        ],
    )(g, g, dp, h, b, Wn)


def _tc_fin(g, dp, h, b, Wc, bc):
    return pl.pallas_call(
        _tc_fin_body,
        grid=(N // BN,),
        in_specs=[
            pl.BlockSpec((BN, HF), lambda i: (i, 0)),
            pl.BlockSpec((BN, HF), lambda i: (N // BN + i, 0)),
            pl.BlockSpec((2, BN, 16), lambda i: (0, i, 0)),
            pl.BlockSpec((BN, D), lambda i: (i, 0)),
            pl.BlockSpec((1, D), lambda i: (0, 0)),
            pl.BlockSpec((D, 64), lambda i: (0, 0)),
            pl.BlockSpec((1, 64), lambda i: (0, 0)),
        ],
        out_specs=pl.BlockSpec((BN, 64), lambda i: (i, 0)),
        out_shape=jax.ShapeDtypeStruct((N, 64), jnp.float32),
    )(g, g, dp, h, b, Wc, bc)


def kernel(x, edge_index, query, W1, b1, W2, b2, W3, b3, W4, b4, Wc, bc):
    src = edge_index[0]
    dst = edge_index[1]
    pad = EP - E
    ar = jnp.arange(pad, dtype=jnp.int32) % DUM
    srcf = jnp.concatenate([src, ar])
    dstf = jnp.concatenate([dst, N + ar])
    srcp2 = jnp.concatenate([srcf, srcf + N]).reshape(-1, 128)
    dst2 = dstf.reshape(-1, 128)
    ones16 = jnp.ones((128, 16), jnp.float32)
    zer16 = jnp.zeros((RZD, 16), jnp.float32)
    zerh = jnp.zeros((DUM, HF), jnp.bfloat16)

    dp = _deg_kernel(dst2, ones16, zer16).reshape(2, N, 16)

    U, h = _tc_pre(x, query, dp, W1)
    g = _spmm_kernel(U.reshape(NC * N, HF), srcp2, dst2, zerh)
    U, h = _tc_mid(g, dp, h, b1.reshape(1, D), W2)
    g = _spmm_kernel(U.reshape(NC * N, HF), srcp2, dst2, zerh)
    U, h = _tc_mid(g, dp, h, b2.reshape(1, D), W3)
    g = _spmm_kernel(U.reshape(NC * N, HF), srcp2, dst2, zerh)
    U, h = _tc_mid(g, dp, h, b3.reshape(1, D), W4)
    g = _spmm_kernel(U.reshape(NC * N, HF), srcp2, dst2, zerh)
    return _tc_fin(g, dp, h, b4.reshape(1, D), Wc,
                   bc.reshape(1, 64))
